# baseline (speedup 1.0000x reference)
import functools

import jax
import jax.numpy as jnp
from jax import lax
from jax.experimental import pallas as pl
from jax.experimental.pallas import tpu as pltpu
from jax.experimental.pallas import tpu_sc as plsc

_P = 196
_PP = 200
_Cv = 128
_V = 100000
_Dnb = 32
_K = 3
_IN = _K * _Dnb + _K
_H = 512
_TILE = 16384
_NT = (_V + _TILE - 1) // _TILE
_NG = 768


def _topk_body(q_ref, t_ref, val_ref, idx_ref, qn_s, t1, t2, t3, i1, i2, i3):
    step = pl.program_id(0)

    @pl.when(step == 0)
    def _init():
        neg = jnp.full((_PP, 128), -jnp.inf, jnp.float32)
        zero = jnp.zeros((_PP, 128), jnp.int32)
        t1[...] = neg
        t2[...] = neg
        t3[...] = neg
        i1[...] = zero
        i2[...] = zero
        i3[...] = zero
        q = q_ref[...]
        qn_s[...] = q / (jnp.sqrt(jnp.sum(q * q, axis=1, keepdims=True))
                         + 1e-12)

    sims = lax.dot_general(
        qn_s[...], t_ref[...], (((1,), (1,)), ((), ())),
        preferred_element_type=jnp.float32, precision=lax.Precision.DEFAULT)

    lane = lax.broadcasted_iota(jnp.int32, (_PP, 128), 1)
    base = step * _TILE
    t1v, t2v, t3v = t1[...], t2[...], t3[...]
    i1v, i2v, i3v = i1[...], i2[...], i3[...]
    for c in range(_TILE // 128):
        v = sims[:, c * 128:(c + 1) * 128]
        if (c + 1) * 128 > _V % _TILE:
            v = jnp.where(lane < _V - base - c * 128, v, -jnp.inf)
        code = step * (_TILE // 128) + c
        c1 = v > t1v
        c2 = v > t2v
        c3 = v > t3v
        m1 = jnp.minimum(t1v, v)
        nt1 = jnp.maximum(t1v, v)
        nt2 = jnp.maximum(t2v, m1)
        nt3 = jnp.maximum(t3v, jnp.minimum(t2v, m1))
        ni1 = jnp.where(c1, code, i1v)
        ni2 = jnp.where(c2, jnp.where(c1, i1v, code), i2v)
        ni3 = jnp.where(c3, jnp.where(c2, i2v, code), i3v)
        t1v, t2v, t3v = nt1, nt2, nt3
        i1v, i2v, i3v = ni1, ni2, ni3
    t1[...] = t1v
    t2[...] = t2v
    t3[...] = t3v
    i1[...] = i1v
    i2[...] = i2v
    i3[...] = i3v

    @pl.when(step == _NT - 1)
    def _merge():
        xs = jnp.concatenate([t1v, t2v, t3v], axis=1)
        gs = jnp.concatenate([i1v * 128 + lane, i2v * 128 + lane,
                              i3v * 128 + lane], axis=1)
        vcols, icols = [], []
        for _ in range(_K):
            m = jnp.max(xs, axis=1, keepdims=True)
            eq = xs == m
            pick = jnp.min(jnp.where(eq, gs, jnp.int32(2**31 - 1)),
                           axis=1, keepdims=True)
            vcols.append(m)
            icols.append(pick)
            xs = jnp.where(eq & (gs == pick), -jnp.inf, xs)
        vpad = jnp.zeros((_PP, 128 - _K), jnp.float32)
        ipad = jnp.zeros((_PP, 128 - _K), jnp.int32)
        val_ref[...] = jnp.concatenate(vcols + [vpad], axis=1)
        idx_ref[...] = jnp.concatenate(icols + [ipad], axis=1)


def _topk_call(q_pad, T_clip):
    return pl.pallas_call(
        _topk_body,
        grid=(_NT,),
        in_specs=[
            pl.BlockSpec((_PP, _Cv), lambda i: (0, 0)),
            pl.BlockSpec((_TILE, _Cv), lambda i: (i, 0)),
        ],
        out_specs=[
            pl.BlockSpec((_PP, 128), lambda i: (0, 0)),
            pl.BlockSpec((_PP, 128), lambda i: (0, 0)),
        ],
        out_shape=[
            jax.ShapeDtypeStruct((_PP, 128), jnp.float32),
            jax.ShapeDtypeStruct((_PP, 128), jnp.int32),
        ],
        scratch_shapes=[pltpu.VMEM((_PP, 128), jnp.float32)] * 4
                      + [pltpu.VMEM((_PP, 128), jnp.int32)] * 3,
        compiler_params=pltpu.CompilerParams(
            dimension_semantics=("arbitrary",)),
    )(q_pad, T_clip)


def _sc_gather(table, flat_idx):
    info = plsc.get_sparse_core_info()
    nc, ns = info.num_cores, info.num_subcores
    nw = nc * ns
    btot = flat_idx.shape[0]
    bw = btot // nw
    width = table.shape[1]
    mesh = plsc.VectorSubcoreMesh(core_axis_name="c", subcore_axis_name="s")

    @functools.partial(
        pl.kernel, mesh=mesh,
        out_type=jax.ShapeDtypeStruct((btot, width), jnp.float32),
        scratch_types=[
            pltpu.VMEM((bw,), jnp.int32),
            pltpu.VMEM((bw, width), jnp.float32),
            pltpu.SemaphoreType.DMA,
        ],
    )
    def gk(table_hbm, idx_hbm, out_hbm, idx_v, rows_v, sem):
        wid = lax.axis_index("s") * nc + lax.axis_index("c")
        base = wid * bw
        pltpu.sync_copy(idx_hbm.at[pl.ds(base, bw)], idx_v)
        pltpu.async_copy(table_hbm.at[idx_v], rows_v, sem).wait()
        pltpu.sync_copy(rows_v, out_hbm.at[pl.ds(base, bw)])

    return gk(table, flat_idx)


def _fuse_body(v_ref, rows_ref, val_ref, idx_ref, lns_ref,
               lnb_ref, w1_ref, b1_ref, w2_ref, b2_ref, wgv_ref, wgn_ref,
               bg_ref, o_ref):
    idx = idx_ref[...]
    nb = []
    for k in range(_K):
        r = rows_ref[256 * k:256 * k + _PP, :]
        sel = lax.rem(idx[:, k:k + 1], 4)
        out = r[:, :_Dnb]
        for s in range(1, 4):
            out = jnp.where(sel == s, r[:, s * _Dnb:(s + 1) * _Dnb], out)
        nb.append(out)
    val = val_ref[...][:, :_K]
    zpad = jnp.zeros((_PP, _Cv - _IN), jnp.float32)
    x = jnp.concatenate(nb + [val, zpad], axis=1)
    lane = lax.broadcasted_iota(jnp.int32, (_PP, 128), 1)
    maskf = (lane < _IN).astype(jnp.float32)
    mu = jnp.sum(x, axis=1, keepdims=True) / _IN
    d = (x - mu) * maskf
    var = jnp.sum(d * d, axis=1, keepdims=True) / _IN
    xh = d / jnp.sqrt(var + 1e-5)
    xh = xh * lns_ref[...] + lnb_ref[...]
    h = lax.dot_general(xh, w1_ref[...], (((1,), (0,)), ((), ())),
                        preferred_element_type=jnp.float32,
                        precision=lax.Precision.HIGHEST)
    h = h + b1_ref[...]
    h = 0.5 * h * (1.0 + lax.erf(h * (2.0 ** -0.5)))
    nb_feat = lax.dot_general(h, w2_ref[...], (((1,), (0,)), ((), ())),
                              preferred_element_type=jnp.float32,
                              precision=lax.Precision.HIGHEST) + b2_ref[...]
    v = v_ref[...]
    gz = (lax.dot_general(v, wgv_ref[...], (((1,), (0,)), ((), ())),
                          preferred_element_type=jnp.float32,
                          precision=lax.Precision.HIGHEST)
          + lax.dot_general(nb_feat, wgn_ref[...], (((1,), (0,)), ((), ())),
                            preferred_element_type=jnp.float32,
                            precision=lax.Precision.HIGHEST)
          + bg_ref[...])
    g = jax.nn.sigmoid(gz)
    f = v + g * nb_feat
    mu2 = jnp.mean(f, axis=1, keepdims=True)
    var2 = jnp.mean((f - mu2) ** 2, axis=1, keepdims=True)
    o_ref[...] = (f - mu2) / jnp.sqrt(var2 + 1e-5)


def _fuse_call(*args):
    return pl.pallas_call(
        _fuse_body,
        out_shape=jax.ShapeDtypeStruct((_PP, _Cv), jnp.float32),
    )(*args)


def kernel(v_seq, T_clip, nb_vecs, ln_scale, ln_bias, W1, b1, W2, b2, Wg, bg):
    q = v_seq[0]
    q_pad = jnp.pad(q, ((0, _PP - _P), (0, 0)))

    vals_p, idx_p = _topk_call(q_pad, T_clip)
    idx = idx_p[:_P, :_K]

    groups = nb_vecs.reshape(_V // 4, 4 * _Dnb)
    flat_idx = jnp.pad((idx >> 2).T, ((0, 0), (0, 60))).reshape(_NG)
    rows = _sc_gather(groups, flat_idx)

    lns = jnp.pad(ln_scale, (0, _Cv - _IN)).reshape(1, _Cv)
    lnb = jnp.pad(ln_bias, (0, _Cv - _IN)).reshape(1, _Cv)
    W1p = jnp.pad(W1, ((0, _Cv - _IN), (0, 0)))
    b1r = b1.reshape(1, _H)
    b2r = b2.reshape(1, _Cv)
    bgr = bg.reshape(1, _Cv)

    out = _fuse_call(q_pad, rows, vals_p, idx_p, lns, lnb, W1p, b1r,
                     W2, b2r, Wg[:_Cv], Wg[_Cv:], bgr)
    return out[:_P].reshape(1, _P, _Cv)


# speedup vs baseline: 1.0397x; 1.0397x over previous
import functools

import jax
import jax.numpy as jnp
from jax import lax
from jax.experimental import pallas as pl
from jax.experimental.pallas import tpu as pltpu
from jax.experimental.pallas import tpu_sc as plsc

_P = 196
_PP = 200
_Cv = 128
_V = 100000
_Dnb = 32
_K = 3
_IN = _K * _Dnb + _K
_H = 512
_TILE = 4096
_NT = (_V + _TILE - 1) // _TILE
_NG = 768


def _topk_body(q_ref, t_ref, val_ref, idx_ref, qn_s, t1, t2, t3, i1, i2, i3):
    step = pl.program_id(0)

    @pl.when(step == 0)
    def _init():
        neg = jnp.full((_PP, 128), -jnp.inf, jnp.float32)
        zero = jnp.zeros((_PP, 128), jnp.int32)
        t1[...] = neg
        t2[...] = neg
        t3[...] = neg
        i1[...] = zero
        i2[...] = zero
        i3[...] = zero
        q = q_ref[...]
        qn_s[...] = q / (jnp.sqrt(jnp.sum(q * q, axis=1, keepdims=True))
                         + 1e-12)

    sims = lax.dot_general(
        qn_s[...], t_ref[...], (((1,), (1,)), ((), ())),
        preferred_element_type=jnp.float32, precision=lax.Precision.DEFAULT)

    lane = lax.broadcasted_iota(jnp.int32, (_PP, 128), 1)
    base = step * _TILE
    t1v, t2v, t3v = t1[...], t2[...], t3[...]
    i1v, i2v, i3v = i1[...], i2[...], i3[...]
    for c in range(_TILE // 128):
        v = sims[:, c * 128:(c + 1) * 128]
        if (c + 1) * 128 > _V % _TILE:
            v = jnp.where(lane < _V - base - c * 128, v, -jnp.inf)
        code = step * (_TILE // 128) + c
        c1 = v > t1v
        c2 = v > t2v
        c3 = v > t3v
        m1 = jnp.minimum(t1v, v)
        nt1 = jnp.maximum(t1v, v)
        nt2 = jnp.maximum(t2v, m1)
        nt3 = jnp.maximum(t3v, jnp.minimum(t2v, m1))
        ni1 = jnp.where(c1, code, i1v)
        ni2 = jnp.where(c2, jnp.where(c1, i1v, code), i2v)
        ni3 = jnp.where(c3, jnp.where(c2, i2v, code), i3v)
        t1v, t2v, t3v = nt1, nt2, nt3
        i1v, i2v, i3v = ni1, ni2, ni3
    t1[...] = t1v
    t2[...] = t2v
    t3[...] = t3v
    i1[...] = i1v
    i2[...] = i2v
    i3[...] = i3v

    @pl.when(step == _NT - 1)
    def _merge():
        xs = jnp.concatenate([t1v, t2v, t3v], axis=1)
        gs = jnp.concatenate([i1v * 128 + lane, i2v * 128 + lane,
                              i3v * 128 + lane], axis=1)
        vcols, icols = [], []
        for _ in range(_K):
            m = jnp.max(xs, axis=1, keepdims=True)
            eq = xs == m
            pick = jnp.min(jnp.where(eq, gs, jnp.int32(2**31 - 1)),
                           axis=1, keepdims=True)
            vcols.append(m)
            icols.append(pick)
            xs = jnp.where(eq & (gs == pick), -jnp.inf, xs)
        vpad = jnp.zeros((_PP, 128 - _K), jnp.float32)
        ipad = jnp.zeros((_PP, 128 - _K), jnp.int32)
        val_ref[...] = jnp.concatenate(vcols + [vpad], axis=1)
        idx_ref[...] = jnp.concatenate(icols + [ipad], axis=1)


def _topk_call(q_pad, T_clip):
    return pl.pallas_call(
        _topk_body,
        grid=(_NT,),
        in_specs=[
            pl.BlockSpec((_PP, _Cv), lambda i: (0, 0)),
            pl.BlockSpec((_TILE, _Cv), lambda i: (i, 0)),
        ],
        out_specs=[
            pl.BlockSpec((_PP, 128), lambda i: (0, 0)),
            pl.BlockSpec((_PP, 128), lambda i: (0, 0)),
        ],
        out_shape=[
            jax.ShapeDtypeStruct((_PP, 128), jnp.float32),
            jax.ShapeDtypeStruct((_PP, 128), jnp.int32),
        ],
        scratch_shapes=[pltpu.VMEM((_PP, 128), jnp.float32)] * 4
                      + [pltpu.VMEM((_PP, 128), jnp.int32)] * 3,
        compiler_params=pltpu.CompilerParams(
            dimension_semantics=("arbitrary",)),
    )(q_pad, T_clip)


def _sc_gather(table, flat_idx):
    info = plsc.get_sparse_core_info()
    nc, ns = info.num_cores, info.num_subcores
    nw = nc * ns
    btot = flat_idx.shape[0]
    bw = btot // nw
    width = table.shape[1]
    mesh = plsc.VectorSubcoreMesh(core_axis_name="c", subcore_axis_name="s")

    @functools.partial(
        pl.kernel, mesh=mesh,
        out_type=jax.ShapeDtypeStruct((btot, width), jnp.float32),
        scratch_types=[
            pltpu.VMEM((bw,), jnp.int32),
            pltpu.VMEM((bw, width), jnp.float32),
            pltpu.SemaphoreType.DMA,
        ],
    )
    def gk(table_hbm, idx_hbm, out_hbm, idx_v, rows_v, sem):
        wid = lax.axis_index("s") * nc + lax.axis_index("c")
        base = wid * bw
        pltpu.sync_copy(idx_hbm.at[pl.ds(base, bw)], idx_v)
        pltpu.async_copy(table_hbm.at[idx_v], rows_v, sem).wait()
        pltpu.sync_copy(rows_v, out_hbm.at[pl.ds(base, bw)])

    return gk(table, flat_idx)


def _fuse_body(v_ref, rows_ref, val_ref, idx_ref, lns_ref,
               lnb_ref, w1_ref, b1_ref, w2_ref, b2_ref, wgv_ref, wgn_ref,
               bg_ref, o_ref):
    idx = idx_ref[...]
    nb = []
    for k in range(_K):
        r = rows_ref[256 * k:256 * k + _PP, :]
        sel = lax.rem(idx[:, k:k + 1], 4)
        out = r[:, :_Dnb]
        for s in range(1, 4):
            out = jnp.where(sel == s, r[:, s * _Dnb:(s + 1) * _Dnb], out)
        nb.append(out)
    val = val_ref[...][:, :_K]
    zpad = jnp.zeros((_PP, _Cv - _IN), jnp.float32)
    x = jnp.concatenate(nb + [val, zpad], axis=1)
    lane = lax.broadcasted_iota(jnp.int32, (_PP, 128), 1)
    maskf = (lane < _IN).astype(jnp.float32)
    mu = jnp.sum(x, axis=1, keepdims=True) / _IN
    d = (x - mu) * maskf
    var = jnp.sum(d * d, axis=1, keepdims=True) / _IN
    xh = d / jnp.sqrt(var + 1e-5)
    xh = xh * lns_ref[...] + lnb_ref[...]
    h = lax.dot_general(xh, w1_ref[...], (((1,), (0,)), ((), ())),
                        preferred_element_type=jnp.float32,
                        precision=lax.Precision.HIGHEST)
    h = h + b1_ref[...]
    h = 0.5 * h * (1.0 + lax.erf(h * (2.0 ** -0.5)))
    nb_feat = lax.dot_general(h, w2_ref[...], (((1,), (0,)), ((), ())),
                              preferred_element_type=jnp.float32,
                              precision=lax.Precision.HIGHEST) + b2_ref[...]
    v = v_ref[...]
    gz = (lax.dot_general(v, wgv_ref[...], (((1,), (0,)), ((), ())),
                          preferred_element_type=jnp.float32,
                          precision=lax.Precision.HIGHEST)
          + lax.dot_general(nb_feat, wgn_ref[...], (((1,), (0,)), ((), ())),
                            preferred_element_type=jnp.float32,
                            precision=lax.Precision.HIGHEST)
          + bg_ref[...])
    g = jax.nn.sigmoid(gz)
    f = v + g * nb_feat
    mu2 = jnp.mean(f, axis=1, keepdims=True)
    var2 = jnp.mean((f - mu2) ** 2, axis=1, keepdims=True)
    o_ref[...] = (f - mu2) / jnp.sqrt(var2 + 1e-5)


def _fuse_call(*args):
    return pl.pallas_call(
        _fuse_body,
        out_shape=jax.ShapeDtypeStruct((_PP, _Cv), jnp.float32),
    )(*args)


def kernel(v_seq, T_clip, nb_vecs, ln_scale, ln_bias, W1, b1, W2, b2, Wg, bg):
    q = v_seq[0]
    q_pad = jnp.pad(q, ((0, _PP - _P), (0, 0)))

    vals_p, idx_p = _topk_call(q_pad, T_clip)
    idx = idx_p[:_P, :_K]

    groups = nb_vecs.reshape(_V // 4, 4 * _Dnb)
    flat_idx = jnp.pad((idx >> 2).T, ((0, 0), (0, 60))).reshape(_NG)
    rows = _sc_gather(groups, flat_idx)

    lns = jnp.pad(ln_scale, (0, _Cv - _IN)).reshape(1, _Cv)
    lnb = jnp.pad(ln_bias, (0, _Cv - _IN)).reshape(1, _Cv)
    W1p = jnp.pad(W1, ((0, _Cv - _IN), (0, 0)))
    b1r = b1.reshape(1, _H)
    b2r = b2.reshape(1, _Cv)
    bgr = bg.reshape(1, _Cv)

    out = _fuse_call(q_pad, rows, vals_p, idx_p, lns, lnb, W1p, b1r,
                     W2, b2r, Wg[:_Cv], Wg[_Cv:], bgr)
    return out[:_P].reshape(1, _P, _Cv)
